# Initial kernel scaffold; baseline (speedup 1.0000x reference)
import jax
import jax.numpy as jnp
from jax.experimental import pallas as pl


def kernel(x_user, x_item, edge_u2i, edge_i2u, params):
    raise NotImplementedError("write your pallas kernel here")


# speedup vs baseline: 22.6856x; 22.6856x over previous
import functools
import math

import jax
import jax.numpy as jnp
from jax import lax
from jax.experimental import pallas as pl
from jax.experimental.pallas import tpu as pltpu
from jax.experimental.pallas import tpu_sc as plsc

N_NODE = 50000
N_EDGE = 300000
HID = 128
OUT = 64

NC = 2
NS = 16
NW = NC * NS

NIDX = 128
EPT = 9728
EPAD = NW * EPT
EPC = EPAD // NS
HP = 16

NCK = 4
CSPAN = 12544
CROW = 12672
RPT = CROW // NS
RCH = 6
RTAIL = RPT - RCH * NIDX


@functools.cache
def _mesh():
  return plsc.VectorSubcoreMesh(
      core_axis_name="c", subcore_axis_name="s", num_cores=NC, num_subcores=NS)


@functools.partial(jax.jit, static_argnames=("width",))
def _sc_gather(table, idx, width):
  ebb = 16384 // width
  nblk = EPT // ebb

  @functools.partial(
      pl.kernel,
      out_type=jax.ShapeDtypeStruct((EPAD, width), jnp.float32),
      mesh=_mesh(),
      scratch_types=[
          pltpu.VMEM((EPT,), jnp.int32),
          [pltpu.VMEM((ebb, width), jnp.float32) for _ in range(4)],
          [pltpu.SemaphoreType.DMA for _ in range(4)],
          [pltpu.SemaphoreType.DMA for _ in range(4)],
      ],
  )
  def k(table_hbm, idx_hbm, out_hbm, idx_all, rows, sem_g, sem_w):
    wid = lax.axis_index("s") * NC + lax.axis_index("c")
    base = wid * EPT
    pltpu.sync_copy(idx_hbm.at[pl.ds(pl.multiple_of(base, 8), EPT)], idx_all)

    def fire(blk, bi):
      pltpu.async_copy(table_hbm.at[idx_all.at[pl.ds(blk * ebb, ebb)]],
                       rows[bi], sem_g[bi])

    def drain(blk, bi):
      pltpu.make_async_copy(table_hbm.at[idx_all.at[pl.ds(blk * ebb, ebb)]],
                            rows[bi], sem_g[bi]).wait()

    fire(0, 0)
    fire(1, 1)

    def wb(blk, bi):
      off = pl.multiple_of(base + blk * ebb, ebb)
      return pltpu.make_async_copy(rows[bi], out_hbm.at[pl.ds(off, ebb)],
                                   sem_w[bi])

    def body(j, _):
      blk0 = j * 4
      for bi in range(4):
        blk = blk0 + bi
        nbi = (bi + 2) % 4

        @pl.when(blk >= 2)
        def _():
          wb(blk - 2, nbi).wait()

        @pl.when(blk + 2 < nblk)
        def _():
          fire(blk + 2, nbi)

        drain(blk, bi)
        off = pl.multiple_of(base + blk * ebb, ebb)
        pltpu.async_copy(rows[bi], out_hbm.at[pl.ds(off, ebb)], sem_w[bi])
      return 0

    lax.fori_loop(0, nblk // 4, body, 0)
    wb(nblk - 2, 2).wait()
    wb(nblk - 1, 3).wait()

  return k(table, idx)


@jax.jit
def _sc_scatter128(vals, di_s, zeros):
  npc = NCK // NC
  nblk = EPC // NIDX

  @functools.partial(
      pl.kernel,
      out_type=jax.ShapeDtypeStruct((NCK, CROW, 128), jnp.float32),
      mesh=_mesh(),
      scratch_types=[
          pltpu.VMEM_SHARED((CROW, 128), jnp.float32),
          pltpu.VMEM((1, NIDX), jnp.int32),
          pltpu.VMEM((NIDX, 128), jnp.float32),
          pltpu.SemaphoreType.DMA,
      ],
  )
  def k(c_hbm, di_hbm, z_hbm, out_hbm, obuf, idx_v, val_v, sem):
    cid = lax.axis_index("c")
    sid = lax.axis_index("s")
    for ch in range(npc):
      ck = cid * npc + ch
      base = ck * CSPAN
      pltpu.sync_copy(z_hbm, val_v)

      def zbody(c, _):
        pltpu.sync_copy(
            val_v, obuf.at[pl.ds(pl.multiple_of(sid * RPT + c * NIDX, 8),
                                 NIDX)])
        return 0

      lax.fori_loop(0, RCH, zbody, 0)
      pltpu.sync_copy(
          val_v.at[pl.ds(0, RTAIL)],
          obuf.at[pl.ds(pl.multiple_of(sid * RPT + RCH * NIDX, 8), RTAIL)])
      plsc.subcore_barrier()

      def body(j, _):
        off = pl.multiple_of(sid * EPC + j * NIDX, NIDX)
        pltpu.sync_copy(di_hbm.at[off // NIDX], idx_v)

        def remap(i, _):
          v = idx_v[0, pl.ds(i * 16, 16)] - base
          ok = (v >= 0) & (v < CSPAN)
          idx_v[0, pl.ds(i * 16, 16)] = jnp.where(ok, v, CSPAN)
          return 0

        lax.fori_loop(0, NIDX // 16, remap, 0)
        pltpu.sync_copy(c_hbm.at[pl.ds(off, NIDX)], val_v)
        pltpu.async_copy(val_v, obuf.at[idx_v.at[0]], sem, add=True).wait()
        return 0

      lax.fori_loop(0, nblk, body, 0)
      plsc.subcore_barrier()

      def dbody(c, _):
        roff = pl.multiple_of(sid * RPT + c * NIDX, 8)
        pltpu.sync_copy(obuf.at[pl.ds(roff, NIDX)], val_v)
        pltpu.sync_copy(val_v, out_hbm.at[ck, pl.ds(roff, NIDX)])
        return 0

      lax.fori_loop(0, RCH, dbody, 0)
      rtoff = pl.multiple_of(sid * RPT + RCH * NIDX, 8)
      pltpu.sync_copy(obuf.at[pl.ds(rtoff, RTAIL)], val_v.at[pl.ds(0, RTAIL)])
      pltpu.sync_copy(val_v.at[pl.ds(0, RTAIL)],
                      out_hbm.at[ck, pl.ds(rtoff, RTAIL)])
      plsc.subcore_barrier()

  return k(vals, di_s.reshape(EPAD // NIDX, 1, NIDX), zeros)


def _assemble(chunks, width):
  return jnp.concatenate(
      [chunks[q, :CSPAN] for q in range(NCK)], axis=0)[:N_NODE, :width]


def _tc_dense(x, w, b, bn, act_out=None):
  n, kdim = x.shape
  m = w.shape[1]

  def body(x_ref, w_ref, b_ref, o_ref):
    y = jnp.dot(x_ref[...], w_ref[...],
                preferred_element_type=jnp.float32) + b_ref[...]
    if act_out == "relu":
      y = jnp.maximum(y, 0.0)
    o_ref[...] = y

  return pl.pallas_call(
      body,
      grid=(n // bn,),
      in_specs=[
          pl.BlockSpec((bn, kdim), lambda i: (i, 0)),
          pl.BlockSpec((kdim, m), lambda i: (0, 0)),
          pl.BlockSpec((1, m), lambda i: (0, 0)),
      ],
      out_specs=pl.BlockSpec((bn, m), lambda i: (i, 0)),
      out_shape=jax.ShapeDtypeStruct((n, m), jnp.float32),
  )(x, w, b.reshape(1, m))


def _tc_proj(x, w, b, bn, width):
  n, kdim = x.shape
  m = w.shape[1]

  def body(x_ref, w_ref, b_ref, q_ref, kv_ref):
    y = jnp.dot(x_ref[...], w_ref[...],
                preferred_element_type=jnp.float32) + b_ref[...]
    q_ref[...] = y[:, :128]
    kv_ref[...] = y[:, 128:]

  return pl.pallas_call(
      body,
      grid=(n // bn,),
      in_specs=[
          pl.BlockSpec((bn, kdim), lambda i: (i, 0)),
          pl.BlockSpec((kdim, m), lambda i: (0, 0)),
          pl.BlockSpec((1, m), lambda i: (0, 0)),
      ],
      out_specs=[
          pl.BlockSpec((bn, 128), lambda i: (i, 0)),
          pl.BlockSpec((bn, 2 * width), lambda i: (i, 0)),
      ],
      out_shape=[
          jax.ShapeDtypeStruct((n, 128), jnp.float32),
          jax.ShapeDtypeStruct((n, 2 * width), jnp.float32),
      ],
  )(x, w, b.reshape(1, m))


def _tc_logits(qg, kvg, smat, width):
  e = qg.shape[0]
  kw2 = kvg.shape[1]
  bn = 2048

  def body(q_ref, kv_ref, s_ref, o_ref):
    t = q_ref[:, :width] * kv_ref[:, :width]
    a = jnp.dot(t, s_ref[...], preferred_element_type=jnp.float32)
    o_ref[...] = jnp.exp(a)

  return pl.pallas_call(
      body,
      grid=(e // bn,),
      in_specs=[
          pl.BlockSpec((bn, 128), lambda i: (i, 0)),
          pl.BlockSpec((bn, kw2), lambda i: (i, 0)),
          pl.BlockSpec((width, 128), lambda i: (0, 0)),
      ],
      out_specs=pl.BlockSpec((bn, 128), lambda i: (i, 0)),
      out_shape=jax.ShapeDtypeStruct((e, 128), jnp.float32),
  )(qg, kvg, smat)


def _tc_contrib(ea, kvg, tmat, pmat, vmask):
  e = ea.shape[0]
  kw2 = kvg.shape[1]
  voff = kw2 - 128
  bn = 2048

  def body(e_ref, kv_ref, t_ref, p_ref, m_ref, o_ref):
    wgt = e_ref[:, :HP]
    msg = jnp.dot(wgt, t_ref[...],
                  preferred_element_type=jnp.float32) * (
                      kv_ref[:, voff:voff + 128] * m_ref[...])
    o_ref[...] = msg + jnp.dot(wgt, p_ref[...],
                               preferred_element_type=jnp.float32)

  return pl.pallas_call(
      body,
      grid=(e // bn,),
      in_specs=[
          pl.BlockSpec((bn, 128), lambda i: (i, 0)),
          pl.BlockSpec((bn, kw2), lambda i: (i, 0)),
          pl.BlockSpec((HP, 128), lambda i: (0, 0)),
          pl.BlockSpec((HP, 128), lambda i: (0, 0)),
          pl.BlockSpec((1, 128), lambda i: (0, 0)),
      ],
      out_specs=pl.BlockSpec((bn, 128), lambda i: (i, 0)),
      out_shape=jax.ShapeDtypeStruct((e, 128), jnp.float32),
  )(ea, kvg, tmat, pmat, vmask)


def _tc_out(o_raw, s, tmat, w, b, bn, res=None, alpha=None):
  n, width = o_raw.shape
  m = w.shape[1]
  ins = [o_raw, s, tmat, w, b.reshape(1, m)]
  specs = [
      pl.BlockSpec((bn, width), lambda i: (i, 0)),
      pl.BlockSpec((bn, HP), lambda i: (i, 0)),
      pl.BlockSpec((HP, width), lambda i: (0, 0)),
      pl.BlockSpec((width, m), lambda i: (0, 0)),
      pl.BlockSpec((1, m), lambda i: (0, 0)),
  ]
  if res is not None:
    ins += [res, alpha.reshape(1, 1)]
    specs += [pl.BlockSpec((bn, m), lambda i: (i, 0)),
              pl.BlockSpec((1, 1), lambda i: (0, 0))]

  def body(*refs):
    o_ref, s_ref, t_ref, w_ref, b_ref = refs[:5]
    out_ref = refs[-1]
    rs = 1.0 / (s_ref[...] + 1e-16)
    oo = o_ref[...] * jnp.dot(rs, t_ref[...],
                              preferred_element_type=jnp.float32)
    xx = 0.5 * oo * (1.0 + lax.erf(oo * (1.0 / math.sqrt(2.0))))
    y = jnp.dot(xx, w_ref[...], preferred_element_type=jnp.float32) + b_ref[...]
    if res is not None:
      y = y + refs[5][...] * refs[6][0, 0]
    out_ref[...] = y

  return pl.pallas_call(
      body,
      grid=(n // bn,),
      in_specs=specs,
      out_specs=pl.BlockSpec((bn, m), lambda i: (i, 0)),
      out_shape=jax.ShapeDtypeStruct((n, m), jnp.float32),
  )(*ins)


def _fold_rel(wmat, bvec, rel, heads, headscale=None):
  kdim = wmat.shape[0]
  d = rel.shape[1]
  wf = jnp.einsum("khd,hdf->khf", wmat.reshape(kdim, heads, d), rel)
  bf = jnp.einsum("hd,hdf->hf", bvec.reshape(heads, d), rel)
  if headscale is not None:
    wf = wf * headscale[None, :, None]
    bf = bf * headscale[:, None]
  return wf.reshape(kdim, heads * d), bf.reshape(heads * d)


def _pad_idx(ix, fill):
  return jnp.concatenate(
      [ix.astype(jnp.int32),
       jnp.full((EPAD - N_EDGE,), fill, jnp.int32)])


def _head_mats(heads, width):
  d = width // heads
  hcol = jnp.arange(HP)
  col = jnp.arange(width) // d
  smat = (col[:, None] == jnp.arange(128)[None, :]).astype(jnp.float32)
  lane_w = jnp.arange(width) // d
  emat = (hcol[:, None] == lane_w[None, :]).astype(jnp.float32)
  if width == 128:
    tmat = emat
    pmat = jnp.zeros((HP, 128), jnp.float32)
    vmask = jnp.ones((1, 128), jnp.float32)
  else:
    lane = jnp.arange(128)
    tmat = (64 + hcol[:, None] * d <= lane[None, :]).astype(jnp.float32) * (
        lane[None, :] < 64 + (hcol[:, None] + 1) * d).astype(jnp.float32)
    pmat = (hcol[:, None] == lane[None, :]).astype(jnp.float32)
    vmask = (lane >= 64).astype(jnp.float32).reshape(1, 128)
  return smat, tmat, pmat, vmask, emat


def _conv(x, p, heads, out_d, eidx, zeros):
  scale = 1.0 / math.sqrt(out_d // heads)
  rel_of = {"user": "u2i", "item": "i2u"}
  smat, tmat, pmat, vmask, emat = _head_mats(heads, out_d)
  proj = {}
  for nt in ("user", "item"):
    et = rel_of[nt]
    wk, bk = _fold_rel(p["Wk"][nt], p["bk"][nt], p["a_rel"][et], heads,
                       headscale=p["p_rel"][et] * scale)
    wv, bv = _fold_rel(p["Wv"][nt], p["bv"][nt], p["m_rel"][et], heads)
    qpad = 128 - out_d
    wq = p["Wq"][nt]
    bq = p["bq"][nt]
    if qpad:
      wq = jnp.concatenate([wq, jnp.zeros((wq.shape[0], qpad), jnp.float32)], 1)
      bq = jnp.concatenate([bq, jnp.zeros((qpad,), jnp.float32)])
    wcat = jnp.concatenate([wq, wk, wv], axis=1)
    bcat = jnp.concatenate([bq, bk, bv])
    proj[nt] = _tc_proj(x[nt], wcat, bcat, 2000, out_d)

  out = {}
  for dst, src, ek in (("item", "user", "u2i"), ("user", "item", "i2u")):
    si, di_g, di_s = eidx[ek]
    kvg = _sc_gather(proj[src][1], si, 2 * out_d)
    qg = _sc_gather(proj[dst][0], di_g, 128)
    ea = _tc_logits(qg, kvg, smat, out_d)
    contrib = _tc_contrib(ea, kvg, tmat, pmat, vmask)
    och = _sc_scatter128(contrib, di_s, zeros)
    if out_d == 128:
      sch = _sc_scatter128(ea, di_s, zeros)
      o_raw = _assemble(och, 128)
      s = _assemble(sch, HP)
    else:
      packed = _assemble(och, 128)
      o_raw = packed[:, 64:64 + out_d]
      s = packed[:, :HP]
    sk = jax.nn.sigmoid(p["skip"][dst])
    if out_d == x[dst].shape[-1]:
      out[dst] = _tc_out(o_raw, s, emat, sk * p["Wa"][dst],
                         sk * p["ba"][dst], 2000, res=x[dst],
                         alpha=(1.0 - sk).reshape(1,))
    else:
      out[dst] = _tc_out(o_raw, s, emat, p["Wa"][dst],
                         p["ba"][dst], 2000)
  return out


def kernel(x_user, x_item, edge_u2i, edge_i2u, params):
  p = params
  zeros = jnp.zeros((NIDX, 128), jnp.float32)
  eidx = {}
  for ek, ei in (("u2i", edge_u2i), ("i2u", edge_i2u)):
    si = _pad_idx(ei[0], 0)
    di_g = _pad_idx(ei[1], 0)
    di_s = _pad_idx(ei[1], N_NODE)
    eidx[ek] = (si, di_g, di_s)

  x = {
      "user": _tc_dense(x_user, p["lin"]["user"]["W"], p["lin"]["user"]["b"],
                        2000, act_out="relu"),
      "item": _tc_dense(x_item, p["lin"]["item"]["W"], p["lin"]["item"]["b"],
                        2000, act_out="relu"),
  }
  x = _conv(x, p["c1"], 8, HID, eidx, zeros)
  x = _conv(x, p["c2"], 4, OUT, eidx, zeros)
  return (x["user"], x["item"])
